# baseline (speedup 1.0000x reference)
import jax
import jax.numpy as jnp
import numpy as _np
from jax import lax
from jax.experimental import pallas as pl
from jax.experimental.pallas import tpu as pltpu
from jax.experimental.pallas import tpu_sc as plsc

N = 10000
D = 128
NPAD = 10240
NW = 32
CH = 128
CHUNKS = 80
EPAD = NW * CHUNKS * CH
RPT = NPAD // 16
IH = 2
HC = CHUNKS // IH
EC = NW * CHUNKS
ECR = 2500

_PAD3 = _np.empty((EC - ECR, 2, CH), _np.int32)
_PAD3[:, 0, :] = (_np.arange((EC - ECR) * CH, dtype=_np.int32).reshape(-1, CH)
                  % (NPAD - N)) + N
_PAD3[:, 1, :] = _PAD3[:, 0, :]

_MESH = plsc.VectorSubcoreMesh(core_axis_name="c", subcore_axis_name="s")


def _deg_body(e_hbm, degp_hbm, idx_d, buf, acc, sem_d):
    cid = lax.axis_index("c")
    sid = lax.axis_index("s")
    wid = cid * 16 + sid
    zeros = jnp.zeros((16,), jnp.float32)
    ones = jnp.full((16,), 1.0, jnp.float32)
    for j in range(8):
        buf[0, pl.ds(j * 16, 16)] = zeros
        buf[1, pl.ds(j * 16, 16)] = ones
    for k in range(RPT // CH):
        pltpu.sync_copy(buf.at[0], acc.at[pl.ds(sid * RPT + k * CH, CH)])
    plsc.subcore_barrier()
    pltpu.sync_copy(e_hbm.at[pl.ds(wid * CHUNKS, CHUNKS)], idx_d)

    def body(c, carry):
        pltpu.async_copy(buf.at[1], acc.at[idx_d.at[c, 1]], sem_d, add=True)
        return carry

    lax.fori_loop(0, CHUNKS, body, 0)

    def dbody(c, carry):
        pltpu.make_async_copy(buf.at[1], acc.at[idx_d.at[c, 1]], sem_d).wait()
        return carry

    lax.fori_loop(0, CHUNKS, dbody, 0)
    plsc.subcore_barrier()
    pltpu.sync_copy(acc.at[pl.ds(sid * RPT, RPT)],
                    degp_hbm.at[cid, pl.ds(sid * RPT, RPT)])


_deg_call = pl.kernel(
    _deg_body,
    out_type=jax.ShapeDtypeStruct((2, NPAD), jnp.float32),
    mesh=_MESH,
    scratch_types=[
        pltpu.VMEM((CHUNKS, 2, CH), jnp.int32),
        pltpu.VMEM((2, CH), jnp.float32),
        pltpu.VMEM_SHARED((NPAD,), jnp.float32),
        pltpu.SemaphoreType.DMA,
    ],
)


def _agg_body(y_hbm, e_hbm, ap_hbm, idx, rows_a, rows_b, acc, sem_a, sem_b):
    cid = lax.axis_index("c")
    sid = lax.axis_index("s")
    wid = cid * 16 + sid
    zeros = jnp.zeros((16,), jnp.float32)

    def zbody(i, carry):
        for j in range(8):
            rows_a[i, pl.ds(j * 16, 16)] = zeros
        return carry

    lax.fori_loop(0, CH, zbody, 0)
    for k in range(RPT // CH):
        pltpu.sync_copy(rows_a, acc.at[pl.ds(sid * RPT + k * CH, CH)])
    plsc.subcore_barrier()

    for h in range(IH):
        pltpu.sync_copy(e_hbm.at[pl.ds(wid * CHUNKS + h * HC, HC)], idx)

        def start_gather(c, rows):
            pltpu.async_copy(y_hbm.at[idx.at[c, 0, pl.ds(0, CH // 2)]],
                             rows.at[pl.ds(0, CH // 2)], sem_a)
            pltpu.async_copy(y_hbm.at[idx.at[c, 0, pl.ds(CH // 2, CH // 2)]],
                             rows.at[pl.ds(CH // 2, CH // 2)], sem_b)

        def wait_gather(c, rows):
            pltpu.make_async_copy(y_hbm.at[idx.at[c, 0, pl.ds(0, CH // 2)]],
                                  rows.at[pl.ds(0, CH // 2)], sem_a).wait()
            pltpu.make_async_copy(y_hbm.at[idx.at[c, 0, pl.ds(CH // 2, CH // 2)]],
                                  rows.at[pl.ds(CH // 2, CH // 2)], sem_b).wait()

        start_gather(0, rows_a)
        start_gather(1, rows_b)

        def body(i, carry):
            c = i * 2
            wait_gather(c, rows_a)
            pltpu.sync_copy(rows_a, acc.at[idx.at[c, 1]], add=True)

            @pl.when(c + 2 < HC)
            def _():
                start_gather(c + 2, rows_a)

            wait_gather(c + 1, rows_b)
            pltpu.sync_copy(rows_b, acc.at[idx.at[c + 1, 1]], add=True)

            @pl.when(c + 3 < HC)
            def _():
                start_gather(c + 3, rows_b)

            return carry

        lax.fori_loop(0, HC // 2, body, 0)
    plsc.subcore_barrier()
    pltpu.sync_copy(acc.at[pl.ds(sid * RPT, RPT)],
                    ap_hbm.at[cid, pl.ds(sid * RPT, RPT)])


_agg_call = pl.kernel(
    _agg_body,
    out_type=jax.ShapeDtypeStruct((2, NPAD, D), jnp.float32),
    mesh=_MESH,
    scratch_types=[
        pltpu.VMEM((HC, 2, CH), jnp.int32),
        pltpu.VMEM((CH, D), jnp.float32),
        pltpu.VMEM((CH, D), jnp.float32),
        pltpu.VMEM_SHARED((NPAD, D), jnp.float32),
        pltpu.SemaphoreType.DMA,
        pltpu.SemaphoreType.DMA,
    ],
)


def _dinv_of(degt_ref):
    deg = degt_ref[:, 0:1] + degt_ref[:, 1:2] + 1.0
    return lax.rsqrt(deg)


def _tc1_body(x_ref, w_ref, degt_ref, y_ref):
    xs = x_ref[...] * _dinv_of(degt_ref)
    y_ref[...] = jnp.dot(xs, w_ref[...], preferred_element_type=jnp.float32)


def _tc2_body(ap_ref, degt_ref, y1_ref, w2_ref, b1_ref, y2_ref):
    dinv = _dinv_of(degt_ref)
    h = jnp.maximum(dinv * (ap_ref[0] + ap_ref[1] + y1_ref[...]) + b1_ref[...],
                    0.0)
    y2_ref[...] = jnp.dot(h * dinv, w2_ref[...],
                          preferred_element_type=jnp.float32)


def _tc3_body(ap_ref, degt_ref, y2_ref, b2_ref, out_ref):
    dinv = _dinv_of(degt_ref)
    out_ref[...] = dinv * (ap_ref[0] + ap_ref[1] + y2_ref[...]) + b2_ref[...]


_BLK = 2000
_GRID = (N // _BLK,)
_f32 = jnp.float32


def _tc1(x, W1, degt):
    return pl.pallas_call(
        _tc1_body,
        grid=_GRID,
        in_specs=[
            pl.BlockSpec((_BLK, D), lambda i: (i, 0)),
            pl.BlockSpec((D, D), lambda i: (0, 0)),
            pl.BlockSpec((_BLK, 2), lambda i: (i, 0)),
        ],
        out_specs=pl.BlockSpec((_BLK, D), lambda i: (i, 0)),
        out_shape=jax.ShapeDtypeStruct((NPAD, D), _f32),
    )(x, W1, degt)


def _tc2(ap1, degt, y1, W2, b1):
    return pl.pallas_call(
        _tc2_body,
        grid=_GRID,
        in_specs=[
            pl.BlockSpec((2, _BLK, D), lambda i: (0, i, 0)),
            pl.BlockSpec((_BLK, 2), lambda i: (i, 0)),
            pl.BlockSpec((_BLK, D), lambda i: (i, 0)),
            pl.BlockSpec((D, D), lambda i: (0, 0)),
            pl.BlockSpec((1, D), lambda i: (0, 0)),
        ],
        out_specs=pl.BlockSpec((_BLK, D), lambda i: (i, 0)),
        out_shape=jax.ShapeDtypeStruct((NPAD, D), _f32),
    )(ap1, degt, y1, W2, b1)


def _tc3(ap2, degt, y2, b2):
    return pl.pallas_call(
        _tc3_body,
        grid=_GRID,
        in_specs=[
            pl.BlockSpec((2, _BLK, D), lambda i: (0, i, 0)),
            pl.BlockSpec((_BLK, 2), lambda i: (i, 0)),
            pl.BlockSpec((_BLK, D), lambda i: (i, 0)),
            pl.BlockSpec((1, D), lambda i: (0, 0)),
        ],
        out_specs=pl.BlockSpec((_BLK, D), lambda i: (i, 0)),
        out_shape=jax.ShapeDtypeStruct((N, D), _f32),
    )(ap2, degt, y2, b2)


def kernel(x, edge_index, W1, b1, W2, b2):
    e3 = jnp.swapaxes(edge_index.reshape(2, ECR, CH), 0, 1)
    e3p = jnp.concatenate([e3, jnp.asarray(_PAD3)], axis=0)

    degp = _deg_call(e3p)
    degt = degp.T
    y1 = _tc1(x, W1, degt)
    ap1 = _agg_call(y1, e3p)
    y2 = _tc2(ap1, degt, y1, W2, b1.reshape(1, D))
    ap2 = _agg_call(y2, e3p)
    return _tc3(ap2, degt, y2, b2.reshape(1, D))


# speedup vs baseline: 1.0194x; 1.0194x over previous
import jax
import jax.numpy as jnp
import numpy as _np
from jax import lax
from jax.experimental import pallas as pl
from jax.experimental.pallas import tpu as pltpu
from jax.experimental.pallas import tpu_sc as plsc

N = 10000
D = 128
NPAD = 10240
NW = 32
CH = 128
CHUNKS = 80
EPAD = NW * CHUNKS * CH
RPT = NPAD // 16
IH = 2
HC = CHUNKS // IH
EC = NW * CHUNKS
ECR = 2500
LASTR = ECR - (NW - 1) * CHUNKS

_PAD3 = _np.empty((EC - ECR, 2, CH), _np.int32)
_PAD3[:, 0, :] = (_np.arange((EC - ECR) * CH, dtype=_np.int32).reshape(-1, CH)
                  % (NPAD - N)) + N
_PAD3[:, 1, :] = _PAD3[:, 0, :]

_MESH = plsc.VectorSubcoreMesh(core_axis_name="c", subcore_axis_name="s")


def _deg_body(e_hbm, pad_hbm, degp_hbm, idx_d, buf, acc, sem_d):
    cid = lax.axis_index("c")
    sid = lax.axis_index("s")
    wid = cid * 16 + sid
    zeros = jnp.zeros((16,), jnp.float32)
    ones = jnp.full((16,), 1.0, jnp.float32)
    for j in range(8):
        buf[0, pl.ds(j * 16, 16)] = zeros
        buf[1, pl.ds(j * 16, 16)] = ones
    for k in range(RPT // CH):
        pltpu.sync_copy(buf.at[0], acc.at[pl.ds(sid * RPT + k * CH, CH)])
    plsc.subcore_barrier()

    @pl.when(wid != NW - 1)
    def _():
        pltpu.sync_copy(e_hbm.at[pl.ds(wid * CHUNKS, CHUNKS)], idx_d)

    @pl.when(wid == NW - 1)
    def _():
        pltpu.sync_copy(e_hbm.at[pl.ds(ECR - LASTR, LASTR)],
                        idx_d.at[pl.ds(0, LASTR)])
        pltpu.sync_copy(pad_hbm, idx_d.at[pl.ds(LASTR, EC - ECR)])

    def body(c, carry):
        pltpu.async_copy(buf.at[1], acc.at[idx_d.at[c, 1]], sem_d, add=True)
        return carry

    lax.fori_loop(0, CHUNKS, body, 0)

    def dbody(c, carry):
        pltpu.make_async_copy(buf.at[1], acc.at[idx_d.at[c, 1]], sem_d).wait()
        return carry

    lax.fori_loop(0, CHUNKS, dbody, 0)
    plsc.subcore_barrier()
    pltpu.sync_copy(acc.at[pl.ds(sid * RPT, RPT)],
                    degp_hbm.at[cid, pl.ds(sid * RPT, RPT)])


_deg_call = pl.kernel(
    _deg_body,
    out_type=jax.ShapeDtypeStruct((2, NPAD), jnp.float32),
    mesh=_MESH,
    scratch_types=[
        pltpu.VMEM((CHUNKS, 2, CH), jnp.int32),
        pltpu.VMEM((2, CH), jnp.float32),
        pltpu.VMEM_SHARED((NPAD,), jnp.float32),
        pltpu.SemaphoreType.DMA,
    ],
)


def _agg_body(y_hbm, e_hbm, pad_hbm, ap_hbm, idx, rows_a, rows_b, acc,
              sem_a, sem_b):
    cid = lax.axis_index("c")
    sid = lax.axis_index("s")
    wid = cid * 16 + sid
    zeros = jnp.zeros((16,), jnp.float32)

    def zbody(i, carry):
        for j in range(8):
            rows_a[i, pl.ds(j * 16, 16)] = zeros
        return carry

    lax.fori_loop(0, CH, zbody, 0)
    for k in range(RPT // CH):
        pltpu.sync_copy(rows_a, acc.at[pl.ds(sid * RPT + k * CH, CH)])
    plsc.subcore_barrier()

    for h in range(IH):
        @pl.when(wid != NW - 1)
        def _():
            pltpu.sync_copy(e_hbm.at[pl.ds(wid * CHUNKS + h * HC, HC)], idx)

        @pl.when(wid == NW - 1)
        def _():
            if h == 0:
                pltpu.sync_copy(e_hbm.at[pl.ds(ECR - LASTR, LASTR)],
                                idx.at[pl.ds(0, LASTR)])
                pltpu.sync_copy(pad_hbm.at[pl.ds(0, HC - LASTR)],
                                idx.at[pl.ds(LASTR, HC - LASTR)])
            else:
                pltpu.sync_copy(pad_hbm.at[pl.ds(HC - LASTR, HC)], idx)

        pltpu.async_copy(y_hbm.at[idx.at[0, 0]], rows_a, sem_a)
        pltpu.async_copy(y_hbm.at[idx.at[1, 0]], rows_b, sem_b)

        def body(i, carry):
            c = i * 2
            pltpu.make_async_copy(y_hbm.at[idx.at[c, 0]], rows_a, sem_a).wait()
            pltpu.sync_copy(rows_a, acc.at[idx.at[c, 1]], add=True)

            @pl.when(c + 2 < HC)
            def _():
                pltpu.async_copy(y_hbm.at[idx.at[c + 2, 0]], rows_a, sem_a)

            pltpu.make_async_copy(y_hbm.at[idx.at[c + 1, 0]], rows_b, sem_b).wait()
            pltpu.sync_copy(rows_b, acc.at[idx.at[c + 1, 1]], add=True)

            @pl.when(c + 3 < HC)
            def _():
                pltpu.async_copy(y_hbm.at[idx.at[c + 3, 0]], rows_b, sem_b)

            return carry

        lax.fori_loop(0, HC // 2, body, 0)
    plsc.subcore_barrier()
    pltpu.sync_copy(acc.at[pl.ds(sid * RPT, RPT)],
                    ap_hbm.at[cid, pl.ds(sid * RPT, RPT)])


_agg_call = pl.kernel(
    _agg_body,
    out_type=jax.ShapeDtypeStruct((2, NPAD, D), jnp.float32),
    mesh=_MESH,
    scratch_types=[
        pltpu.VMEM((HC, 2, CH), jnp.int32),
        pltpu.VMEM((CH, D), jnp.float32),
        pltpu.VMEM((CH, D), jnp.float32),
        pltpu.VMEM_SHARED((NPAD, D), jnp.float32),
        pltpu.SemaphoreType.DMA,
        pltpu.SemaphoreType.DMA,
    ],
)


def _dinv_of(degt_ref):
    deg = degt_ref[:, 0:1] + degt_ref[:, 1:2] + 1.0
    return lax.rsqrt(deg)


def _tc1_body(x_ref, w_ref, degt_ref, y_ref):
    xs = x_ref[...] * _dinv_of(degt_ref)
    y_ref[...] = jnp.dot(xs, w_ref[...], preferred_element_type=jnp.float32)


def _tc2_body(ap_ref, degt_ref, y1_ref, w2_ref, b1_ref, y2_ref):
    dinv = _dinv_of(degt_ref)
    h = jnp.maximum(dinv * (ap_ref[0] + ap_ref[1] + y1_ref[...]) + b1_ref[...],
                    0.0)
    y2_ref[...] = jnp.dot(h * dinv, w2_ref[...],
                          preferred_element_type=jnp.float32)


def _tc3_body(ap_ref, degt_ref, y2_ref, b2_ref, out_ref):
    dinv = _dinv_of(degt_ref)
    out_ref[...] = dinv * (ap_ref[0] + ap_ref[1] + y2_ref[...]) + b2_ref[...]


_BLK = 2000
_GRID = (N // _BLK,)
_f32 = jnp.float32


def _tc1(x, W1, degt):
    return pl.pallas_call(
        _tc1_body,
        grid=_GRID,
        in_specs=[
            pl.BlockSpec((_BLK, D), lambda i: (i, 0)),
            pl.BlockSpec((D, D), lambda i: (0, 0)),
            pl.BlockSpec((_BLK, 2), lambda i: (i, 0)),
        ],
        out_specs=pl.BlockSpec((_BLK, D), lambda i: (i, 0)),
        out_shape=jax.ShapeDtypeStruct((NPAD, D), _f32),
    )(x, W1, degt)


def _tc2(ap1, degt, y1, W2, b1):
    return pl.pallas_call(
        _tc2_body,
        grid=_GRID,
        in_specs=[
            pl.BlockSpec((2, _BLK, D), lambda i: (0, i, 0)),
            pl.BlockSpec((_BLK, 2), lambda i: (i, 0)),
            pl.BlockSpec((_BLK, D), lambda i: (i, 0)),
            pl.BlockSpec((D, D), lambda i: (0, 0)),
            pl.BlockSpec((1, D), lambda i: (0, 0)),
        ],
        out_specs=pl.BlockSpec((_BLK, D), lambda i: (i, 0)),
        out_shape=jax.ShapeDtypeStruct((NPAD, D), _f32),
    )(ap1, degt, y1, W2, b1)


def _tc3(ap2, degt, y2, b2):
    return pl.pallas_call(
        _tc3_body,
        grid=_GRID,
        in_specs=[
            pl.BlockSpec((2, _BLK, D), lambda i: (0, i, 0)),
            pl.BlockSpec((_BLK, 2), lambda i: (i, 0)),
            pl.BlockSpec((_BLK, D), lambda i: (i, 0)),
            pl.BlockSpec((1, D), lambda i: (0, 0)),
        ],
        out_specs=pl.BlockSpec((_BLK, D), lambda i: (i, 0)),
        out_shape=jax.ShapeDtypeStruct((N, D), _f32),
    )(ap2, degt, y2, b2)


def kernel(x, edge_index, W1, b1, W2, b2):
    e3 = jnp.swapaxes(edge_index.reshape(2, ECR, CH), 0, 1)
    pad3 = jnp.asarray(_PAD3)

    degp = _deg_call(e3, pad3)
    degt = degp.T
    y1 = _tc1(x, W1, degt)
    ap1 = _agg_call(y1, e3, pad3)
    y2 = _tc2(ap1, degt, y1, W2, b1.reshape(1, D))
    ap2 = _agg_call(y2, e3, pad3)
    return _tc3(ap2, degt, y2, b2.reshape(1, D))


# speedup vs baseline: 1.0306x; 1.0110x over previous
import jax
import jax.numpy as jnp
from jax import lax
from jax.experimental import pallas as pl
from jax.experimental.pallas import tpu as pltpu
from jax.experimental.pallas import tpu_sc as plsc

N = 10000
D = 128
NPAD = 10240
NW = 32
CH = 128
CHUNKS = 80
EPAD = NW * CHUNKS * CH
RPT = NPAD // 16
IH = 2
HC = CHUNKS // IH
EC = NW * CHUNKS
ECR = 2500
LASTR = ECR - (NW - 1) * CHUNKS

_MESH = plsc.VectorSubcoreMesh(core_axis_name="c", subcore_axis_name="s")


def _deg_body(e_hbm, degp_hbm, idx_d, buf, acc, sem_d):
    cid = lax.axis_index("c")
    sid = lax.axis_index("s")
    wid = cid * 16 + sid
    zeros = jnp.zeros((16,), jnp.float32)
    ones = jnp.full((16,), 1.0, jnp.float32)
    for j in range(8):
        buf[0, pl.ds(j * 16, 16)] = zeros
        buf[1, pl.ds(j * 16, 16)] = ones
    for k in range(RPT // CH):
        pltpu.sync_copy(buf.at[0], acc.at[pl.ds(sid * RPT + k * CH, CH)])
    plsc.subcore_barrier()
    nc = jnp.where(wid == NW - 1, LASTR, CHUNKS)

    @pl.when(wid != NW - 1)
    def _():
        pltpu.sync_copy(e_hbm.at[pl.ds(wid * CHUNKS, CHUNKS)], idx_d)

    @pl.when(wid == NW - 1)
    def _():
        pltpu.sync_copy(e_hbm.at[pl.ds(ECR - LASTR, LASTR)],
                        idx_d.at[pl.ds(0, LASTR)])

    def body(c, carry):
        pltpu.async_copy(buf.at[1], acc.at[idx_d.at[c, 1]], sem_d, add=True)
        return carry

    lax.fori_loop(0, nc, body, 0)

    def dbody(c, carry):
        pltpu.make_async_copy(buf.at[1], acc.at[idx_d.at[c, 1]], sem_d).wait()
        return carry

    lax.fori_loop(0, nc, dbody, 0)
    plsc.subcore_barrier()
    pltpu.sync_copy(acc.at[pl.ds(sid * RPT, RPT)],
                    degp_hbm.at[cid, pl.ds(sid * RPT, RPT)])


_deg_call = pl.kernel(
    _deg_body,
    out_type=jax.ShapeDtypeStruct((2, NPAD), jnp.float32),
    mesh=_MESH,
    scratch_types=[
        pltpu.VMEM((CHUNKS, 2, CH), jnp.int32),
        pltpu.VMEM((2, CH), jnp.float32),
        pltpu.VMEM_SHARED((NPAD,), jnp.float32),
        pltpu.SemaphoreType.DMA,
    ],
)


def _agg_body(y_hbm, e_hbm, ap_hbm, idx, rows_a, rows_b, acc, sem_a, sem_b):
    cid = lax.axis_index("c")
    sid = lax.axis_index("s")
    wid = cid * 16 + sid
    zeros = jnp.zeros((16,), jnp.float32)

    def zbody(i, carry):
        for j in range(8):
            rows_a[i, pl.ds(j * 16, 16)] = zeros
        return carry

    lax.fori_loop(0, CH, zbody, 0)
    for k in range(RPT // CH):
        pltpu.sync_copy(rows_a, acc.at[pl.ds(sid * RPT + k * CH, CH)])
    plsc.subcore_barrier()

    for h in range(IH):
        hcw = jnp.where(wid == NW - 1, LASTR if h == 0 else 0, HC)

        @pl.when(wid != NW - 1)
        def _():
            pltpu.sync_copy(e_hbm.at[pl.ds(wid * CHUNKS + h * HC, HC)], idx)

        if h == 0:
            @pl.when(wid == NW - 1)
            def _():
                pltpu.sync_copy(e_hbm.at[pl.ds(ECR - LASTR, LASTR)],
                                idx.at[pl.ds(0, LASTR)])

        @pl.when(hcw > 0)
        def _():
            pltpu.async_copy(y_hbm.at[idx.at[0, 0]], rows_a, sem_a)
            pltpu.async_copy(y_hbm.at[idx.at[1, 0]], rows_b, sem_b)

        def body(i, carry):
            c = i * 2
            pltpu.make_async_copy(y_hbm.at[idx.at[c, 0]], rows_a, sem_a).wait()
            pltpu.sync_copy(rows_a, acc.at[idx.at[c, 1]], add=True)

            @pl.when(c + 2 < hcw)
            def _():
                pltpu.async_copy(y_hbm.at[idx.at[c + 2, 0]], rows_a, sem_a)

            pltpu.make_async_copy(y_hbm.at[idx.at[c + 1, 0]], rows_b, sem_b).wait()
            pltpu.sync_copy(rows_b, acc.at[idx.at[c + 1, 1]], add=True)

            @pl.when(c + 3 < hcw)
            def _():
                pltpu.async_copy(y_hbm.at[idx.at[c + 3, 0]], rows_b, sem_b)

            return carry

        lax.fori_loop(0, hcw // 2, body, 0)
    plsc.subcore_barrier()
    pltpu.sync_copy(acc.at[pl.ds(sid * RPT, RPT)],
                    ap_hbm.at[cid, pl.ds(sid * RPT, RPT)])


_agg_call = pl.kernel(
    _agg_body,
    out_type=jax.ShapeDtypeStruct((2, NPAD, D), jnp.float32),
    mesh=_MESH,
    scratch_types=[
        pltpu.VMEM((HC, 2, CH), jnp.int32),
        pltpu.VMEM((CH, D), jnp.float32),
        pltpu.VMEM((CH, D), jnp.float32),
        pltpu.VMEM_SHARED((NPAD, D), jnp.float32),
        pltpu.SemaphoreType.DMA,
        pltpu.SemaphoreType.DMA,
    ],
)


def _dinv_of(degt_ref):
    deg = degt_ref[:, 0:1] + degt_ref[:, 1:2] + 1.0
    return lax.rsqrt(deg)


def _tc1_body(x_ref, w_ref, degt_ref, y_ref):
    xs = x_ref[...] * _dinv_of(degt_ref)
    y_ref[...] = jnp.dot(xs, w_ref[...], preferred_element_type=jnp.float32)


def _tc2_body(ap_ref, degt_ref, y1_ref, w2_ref, b1_ref, y2_ref):
    dinv = _dinv_of(degt_ref)
    h = jnp.maximum(dinv * (ap_ref[0] + ap_ref[1] + y1_ref[...]) + b1_ref[...],
                    0.0)
    y2_ref[...] = jnp.dot(h * dinv, w2_ref[...],
                          preferred_element_type=jnp.float32)


def _tc3_body(ap_ref, degt_ref, y2_ref, b2_ref, out_ref):
    dinv = _dinv_of(degt_ref)
    out_ref[...] = dinv * (ap_ref[0] + ap_ref[1] + y2_ref[...]) + b2_ref[...]


_BLK = 2000
_GRID = (N // _BLK,)
_f32 = jnp.float32


def _tc1(x, W1, degt):
    return pl.pallas_call(
        _tc1_body,
        grid=_GRID,
        in_specs=[
            pl.BlockSpec((_BLK, D), lambda i: (i, 0)),
            pl.BlockSpec((D, D), lambda i: (0, 0)),
            pl.BlockSpec((_BLK, 2), lambda i: (i, 0)),
        ],
        out_specs=pl.BlockSpec((_BLK, D), lambda i: (i, 0)),
        out_shape=jax.ShapeDtypeStruct((NPAD, D), _f32),
    )(x, W1, degt)


def _tc2(ap1, degt, y1, W2, b1):
    return pl.pallas_call(
        _tc2_body,
        grid=_GRID,
        in_specs=[
            pl.BlockSpec((2, _BLK, D), lambda i: (0, i, 0)),
            pl.BlockSpec((_BLK, 2), lambda i: (i, 0)),
            pl.BlockSpec((_BLK, D), lambda i: (i, 0)),
            pl.BlockSpec((D, D), lambda i: (0, 0)),
            pl.BlockSpec((1, D), lambda i: (0, 0)),
        ],
        out_specs=pl.BlockSpec((_BLK, D), lambda i: (i, 0)),
        out_shape=jax.ShapeDtypeStruct((NPAD, D), _f32),
    )(ap1, degt, y1, W2, b1)


def _tc3(ap2, degt, y2, b2):
    return pl.pallas_call(
        _tc3_body,
        grid=_GRID,
        in_specs=[
            pl.BlockSpec((2, _BLK, D), lambda i: (0, i, 0)),
            pl.BlockSpec((_BLK, 2), lambda i: (i, 0)),
            pl.BlockSpec((_BLK, D), lambda i: (i, 0)),
            pl.BlockSpec((1, D), lambda i: (0, 0)),
        ],
        out_specs=pl.BlockSpec((_BLK, D), lambda i: (i, 0)),
        out_shape=jax.ShapeDtypeStruct((N, D), _f32),
    )(ap2, degt, y2, b2)


def kernel(x, edge_index, W1, b1, W2, b2):
    e3 = jnp.swapaxes(edge_index.reshape(2, ECR, CH), 0, 1)

    degp = _deg_call(e3)
    degt = degp.T
    y1 = _tc1(x, W1, degt)
    ap1 = _agg_call(y1, e3)
    y2 = _tc2(ap1, degt, y1, W2, b1.reshape(1, D))
    ap2 = _agg_call(y2, e3)
    return _tc3(ap2, degt, y2, b2.reshape(1, D))
